# baseline (speedup 1.0000x reference)
import jax
import jax.numpy as jnp
from jax.experimental import pallas as pl
from jax.experimental.pallas import tpu as pltpu

_BM = 200
_NBUF = 4
_SUP_CHUNK = 2000


def _gcn_kernel(x_ref, adj_hbm, w_ref, out_ref, sup_ref, bufs_ref, sems):
    i = pl.program_id(0)
    nsteps = pl.num_programs(0)

    def start_copy(strip):
        slot = jax.lax.rem(strip, _NBUF)
        pltpu.make_async_copy(
            adj_hbm.at[pl.ds(strip * _BM, _BM), :],
            bufs_ref.at[slot],
            sems.at[slot],
        ).start()

    @pl.when(i == 0)
    def _():
        for b in range(_NBUF - 1):
            start_copy(jnp.int32(b))

        w = w_ref[...]

        def body(c, _):
            xc = x_ref[pl.ds(c * _SUP_CHUNK, _SUP_CHUNK), :]
            sup_ref[pl.ds(c * _SUP_CHUNK, _SUP_CHUNK), :] = jax.lax.dot_general(
                xc, w,
                dimension_numbers=(((1,), (0,)), ((), ())),
                preferred_element_type=jnp.float32,
            )
            return _

        jax.lax.fori_loop(0, x_ref.shape[0] // _SUP_CHUNK, body, None)

    ahead = i + _NBUF - 1

    @pl.when(ahead < nsteps)
    def _():
        start_copy(ahead)

    slot = jax.lax.rem(i, _NBUF)
    pltpu.make_async_copy(
        adj_hbm.at[pl.ds(i * _BM, _BM), :],
        bufs_ref.at[slot],
        sems.at[slot],
    ).wait()

    out_ref[...] = jax.lax.dot_general(
        bufs_ref[slot], sup_ref[...],
        dimension_numbers=(((1,), (0,)), ((), ())),
        preferred_element_type=jnp.float32,
    )


def kernel(x, adj, weight):
    n, din = x.shape
    dout = weight.shape[1]

    return pl.pallas_call(
        _gcn_kernel,
        grid=(n // _BM,),
        in_specs=[
            pl.BlockSpec((n, din), lambda i: (0, 0)),
            pl.BlockSpec(memory_space=pl.ANY),
            pl.BlockSpec((din, dout), lambda i: (0, 0)),
        ],
        out_specs=pl.BlockSpec((_BM, dout), lambda i: (i, 0)),
        out_shape=jax.ShapeDtypeStruct((n, dout), jnp.float32),
        scratch_shapes=[
            pltpu.VMEM((n, dout), jnp.float32),
            pltpu.VMEM((_NBUF, _BM, n), jnp.float32),
            pltpu.SemaphoreType.DMA((_NBUF,)),
        ],
        compiler_params=pltpu.CompilerParams(
            dimension_semantics=("arbitrary",),
        ),
    )(x, adj, weight)


# speedup vs baseline: 1.0187x; 1.0187x over previous
import jax
import jax.numpy as jnp
from jax.experimental import pallas as pl
from jax.experimental.pallas import tpu as pltpu

_BM = 200
_NBUF = 3
_SUP_CHUNK = 2000


def _gcn_kernel(x_ref, adj_hbm, w_ref, out_ref, sup_ref, bufs_ref, sems):
    i = pl.program_id(0)
    nsteps = pl.num_programs(0)

    def start_copy(strip):
        slot = jax.lax.rem(strip, _NBUF)
        pltpu.make_async_copy(
            adj_hbm.at[pl.ds(strip * _BM, _BM), :],
            bufs_ref.at[slot],
            sems.at[slot],
        ).start()

    @pl.when(i == 0)
    def _():
        for b in range(_NBUF - 1):
            start_copy(jnp.int32(b))

        w = w_ref[...]

        def body(c, _):
            xc = x_ref[pl.ds(c * _SUP_CHUNK, _SUP_CHUNK), :]
            sup_ref[pl.ds(c * _SUP_CHUNK, _SUP_CHUNK), :] = jax.lax.dot_general(
                xc, w,
                dimension_numbers=(((1,), (0,)), ((), ())),
                preferred_element_type=jnp.float32,
            )
            return _

        jax.lax.fori_loop(0, x_ref.shape[0] // _SUP_CHUNK, body, None)

    ahead = i + _NBUF - 1

    @pl.when(ahead < nsteps)
    def _():
        start_copy(ahead)

    slot = jax.lax.rem(i, _NBUF)
    pltpu.make_async_copy(
        adj_hbm.at[pl.ds(i * _BM, _BM), :],
        bufs_ref.at[slot],
        sems.at[slot],
    ).wait()

    out_ref[...] = jax.lax.dot_general(
        bufs_ref[slot], sup_ref[...],
        dimension_numbers=(((1,), (0,)), ((), ())),
        preferred_element_type=jnp.float32,
    )


def kernel(x, adj, weight):
    n, din = x.shape
    dout = weight.shape[1]

    return pl.pallas_call(
        _gcn_kernel,
        grid=(n // _BM,),
        in_specs=[
            pl.BlockSpec((n, din), lambda i: (0, 0)),
            pl.BlockSpec(memory_space=pl.ANY),
            pl.BlockSpec((din, dout), lambda i: (0, 0)),
        ],
        out_specs=pl.BlockSpec((_BM, dout), lambda i: (i, 0)),
        out_shape=jax.ShapeDtypeStruct((n, dout), jnp.float32),
        scratch_shapes=[
            pltpu.VMEM((n, dout), jnp.float32),
            pltpu.VMEM((_NBUF, _BM, n), jnp.float32),
            pltpu.SemaphoreType.DMA((_NBUF,)),
        ],
        compiler_params=pltpu.CompilerParams(
            dimension_semantics=("arbitrary",),
        ),
    )(x, adj, weight)


# speedup vs baseline: 1.0193x; 1.0006x over previous
import jax
import jax.numpy as jnp
from jax.experimental import pallas as pl
from jax.experimental.pallas import tpu as pltpu

_BM = 200
_NBUF = 3
_SUP_CHUNK = 2000


def _gcn_kernel(x_ref, adj_hbm, w_ref, out_ref, sup_ref, bufs_ref, sems):
    i = pl.program_id(0)
    nsteps = pl.num_programs(0)

    splits = ((0, 96), (96, 104))

    def start_copy(strip):
        slot = jax.lax.rem(strip, _NBUF)
        for h, (off, sz) in enumerate(splits):
            pltpu.make_async_copy(
                adj_hbm.at[pl.ds(strip * _BM + off, sz), :],
                bufs_ref.at[slot, pl.ds(off, sz)],
                sems.at[slot, h],
            ).start()

    @pl.when(i == 0)
    def _():
        for b in range(_NBUF - 1):
            start_copy(jnp.int32(b))

        w = w_ref[...]

        def body(c, _):
            xc = x_ref[pl.ds(c * _SUP_CHUNK, _SUP_CHUNK), :]
            sup_ref[pl.ds(c * _SUP_CHUNK, _SUP_CHUNK), :] = jax.lax.dot_general(
                xc, w,
                dimension_numbers=(((1,), (0,)), ((), ())),
                preferred_element_type=jnp.float32,
            )
            return _

        jax.lax.fori_loop(0, x_ref.shape[0] // _SUP_CHUNK, body, None)

    ahead = i + _NBUF - 1

    @pl.when(ahead < nsteps)
    def _():
        start_copy(ahead)

    slot = jax.lax.rem(i, _NBUF)
    for h, (off, sz) in enumerate(splits):
        pltpu.make_async_copy(
            adj_hbm.at[pl.ds(i * _BM + off, sz), :],
            bufs_ref.at[slot, pl.ds(off, sz)],
            sems.at[slot, h],
        ).wait()

    out_ref[...] = jax.lax.dot_general(
        bufs_ref[slot], sup_ref[...],
        dimension_numbers=(((1,), (0,)), ((), ())),
        preferred_element_type=jnp.float32,
    )


def kernel(x, adj, weight):
    n, din = x.shape
    dout = weight.shape[1]

    return pl.pallas_call(
        _gcn_kernel,
        grid=(n // _BM,),
        in_specs=[
            pl.BlockSpec((n, din), lambda i: (0, 0)),
            pl.BlockSpec(memory_space=pl.ANY),
            pl.BlockSpec((din, dout), lambda i: (0, 0)),
        ],
        out_specs=pl.BlockSpec((_BM, dout), lambda i: (i, 0)),
        out_shape=jax.ShapeDtypeStruct((n, dout), jnp.float32),
        scratch_shapes=[
            pltpu.VMEM((n, dout), jnp.float32),
            pltpu.VMEM((_NBUF, _BM, n), jnp.float32),
            pltpu.SemaphoreType.DMA((_NBUF, 2)),
        ],
        compiler_params=pltpu.CompilerParams(
            dimension_semantics=("arbitrary",),
        ),
    )(x, adj, weight)
